# baseline (speedup 1.0000x reference)
import functools

import jax
import jax.numpy as jnp
from jax import lax
from jax.experimental import pallas as pl
from jax.experimental.pallas import tpu as pltpu
from jax.experimental.pallas import tpu_sc as plsc

_B, _S, _D = 4, 8192, 768
_MASK_RATIO = 0.15
_NUM_MASK = int(_S * _MASK_RATIO)
_P = 1280
_MBUF = _S + 64
_KC = 512
_BS = 1024


def _ranks_body(noise_ref, pivt_ref, out_ref):
    sub_iota = lax.broadcasted_iota(jnp.int32, (_P, 1), 0)
    lane_iota = lax.broadcasted_iota(jnp.int32, (1, _P), 1)
    for b in range(_B):
        piv_col = pivt_ref[:, b : b + 1]
        piv_col = jnp.where(sub_iota >= _NUM_MASK, jnp.inf, piv_col)
        acc = jnp.zeros((_P, 1), jnp.float32)
        for c in range(_S // _KC):
            chunk = noise_ref[b : b + 1, c * _KC : (c + 1) * _KC]
            lt = (chunk < piv_col).astype(jnp.float32)
            acc = acc + jnp.sum(lt, axis=1, keepdims=True)
        piv_row = noise_ref[b : b + 1, :_P]
        piv_row = jnp.where(lane_iota >= _NUM_MASK, jnp.inf, piv_row)
        eq = (piv_row == piv_col) & (lane_iota < sub_iota)
        acc = acc + jnp.sum(eq.astype(jnp.float32), axis=1, keepdims=True)
        out_ref[:, b : b + 1] = acc.astype(jnp.int32)


def _mask_sc_body(ranks_hbm, mask_hbm, idx_v, row_v):
    wid = lax.axis_index("s") * 2 + lax.axis_index("c")

    @pl.when(wid < _B)
    def _():
        pltpu.sync_copy(ranks_hbm.at[wid], idx_v)
        ones = jnp.ones((16,), jnp.float32)
        zeros = jnp.zeros((16,), jnp.float32)

        def _init(i, carry):
            row_v[pl.ds(i * 16, 16)] = ones
            return carry

        lax.fori_loop(0, _MBUF // 16, _init, 0)

        def _scat(i, carry):
            idx = idx_v[pl.ds(i * 16, 16)]
            plsc.store_scatter(row_v, [idx], zeros)
            return carry

        lax.fori_loop(0, _P // 16, _scat, 0)
        pltpu.sync_copy(row_v.at[pl.ds(0, _S)], mask_hbm.at[wid])


def _apply_body(mask_ref, x_ref, out_ref, loss_ref, acc_ref):
    b = pl.program_id(0)
    j = pl.program_id(1)

    @pl.when((b == 0) & (j == 0))
    def _():
        acc_ref[0] = 0.0
        acc_ref[1] = 0.0

    m = mask_ref[0]
    xb = x_ref[0]
    xm = xb * m
    out_ref[0] = xm
    d = xm - xb
    acc_ref[0] += jnp.sum(d * d * m)
    acc_ref[1] += jnp.sum(m)

    @pl.when((b == _B - 1) & (j == pl.num_programs(1) - 1))
    def _():
        val = acc_ref[0] / jnp.float32(_D) / acc_ref[1]
        loss_ref[...] = jnp.full((1, 1), val, jnp.float32)


def _build_mask_sc():
    mesh = plsc.VectorSubcoreMesh(core_axis_name="c", subcore_axis_name="s")
    return pl.kernel(
        _mask_sc_body,
        out_type=jax.ShapeDtypeStruct((_B, _S), jnp.float32),
        mesh=mesh,
        scratch_types=[
            pltpu.VMEM((_P,), jnp.int32),
            pltpu.VMEM((_MBUF,), jnp.float32),
        ],
        compiler_params=pltpu.CompilerParams(needs_layout_passes=False),
    )


@jax.jit
def kernel(x, noise):
    piv_t = noise[:, :_P].T
    ranks_t = pl.pallas_call(
        _ranks_body,
        out_shape=jax.ShapeDtypeStruct((_P, _B), jnp.int32),
    )(noise, piv_t)
    ranks = ranks_t.T

    mask = _build_mask_sc()(ranks)

    mask3 = mask[:, :, None]
    pred, loss = pl.pallas_call(
        _apply_body,
        grid=(_B, _S // _BS),
        in_specs=[
            pl.BlockSpec((1, _BS, 1), lambda b, j: (b, j, 0)),
            pl.BlockSpec((1, _BS, _D), lambda b, j: (b, j, 0)),
        ],
        out_specs=[
            pl.BlockSpec((1, _BS, _D), lambda b, j: (b, j, 0)),
            pl.BlockSpec((1, 1), lambda b, j: (0, 0)),
        ],
        out_shape=[
            jax.ShapeDtypeStruct((_B, _S, _D), jnp.float32),
            jax.ShapeDtypeStruct((1, 1), jnp.float32),
        ],
        scratch_shapes=[pltpu.SMEM((2,), jnp.float32)],
    )(mask3, x)
    return pred, mask, loss[0, 0]
